# baseline (speedup 1.0000x reference)
import functools

import jax
import jax.numpy as jnp
from jax import lax
from jax.experimental import pallas as pl
from jax.experimental.pallas import tpu as pltpu
from jax.experimental.pallas import tpu_sc as plsc

N = 10000
E = 160000
ND = 256
ERD = 16
EAD = 16
ACD = ND + ERD + EAD
HD0 = 176
ZP = 256
HP = 128

NC = 2
NS = 16
NW = NC * NS

_POFF = (0, 64000, 128000)
_PLEN = (64000, 64000, 32000)
_CH = 200
_TB = 3200

_NRPS = 640
_NPAD = NS * _NRPS

_vmesh = plsc.VectorSubcoreMesh(core_axis_name="c", subcore_axis_name="s")


def _mk_gather_compose(off, ln):
    epw = ln // NW
    nch = epw // _CH

    @functools.partial(
        pl.kernel,
        mesh=_vmesh,
        out_type=jax.ShapeDtypeStruct((ln, ND), jnp.float32),
        scratch_types=[
            pltpu.VMEM((1, _CH), jnp.int32),
            pltpu.VMEM((1, _CH), jnp.int32),
            pltpu.VMEM((_CH, ND), jnp.float32),
        ],
    )
    def _gather_compose(nf_hbm, src2d_hbm, srcfull_hbm, x2_hbm,
                        srcv, src2v, rows):
        wid = lax.axis_index("s") * NC + lax.axis_index("c")

        @pl.loop(0, nch)
        def _(k):
            crow = (off + wid * epw) // _CH + k
            pltpu.sync_copy(src2d_hbm.at[pl.ds(crow, 1)], srcv)
            pltpu.sync_copy(srcfull_hbm.at[srcv.at[0]], src2v.at[0])
            pltpu.sync_copy(nf_hbm.at[src2v.at[0]], rows)
            pltpu.sync_copy(rows, x2_hbm.at[pl.ds(wid * epw + k * _CH, _CH)])

    return _gather_compose


def _mk_scatter_add(off, ln):
    eps = ln // NS
    nch = eps // _CH

    @functools.partial(
        pl.kernel,
        mesh=_vmesh,
        out_type=[
            jax.ShapeDtypeStruct((_NPAD, HP), jnp.float32),
            jax.ShapeDtypeStruct((_NPAD, HP), jnp.float32),
        ],
        scratch_types=[
            pltpu.VMEM((1, _CH), jnp.int32),
            pltpu.VMEM((_CH, HP), jnp.float32),
            pltpu.VMEM_SHARED((_NPAD, HP), jnp.float32),
        ],
    )
    def _scatter_add(z_hbm, dst2d_hbm, zeros_hbm, m0_hbm, m1_hbm,
                     dstv, rows, acc):
        cid = lax.axis_index("c")
        sid = lax.axis_index("s")

        pltpu.sync_copy(zeros_hbm, acc.at[pl.ds(sid * _NRPS, _NRPS)])
        plsc.subcore_barrier()

        @pl.loop(0, nch)
        def _(k):
            base = sid * eps + k * _CH
            pltpu.sync_copy(dst2d_hbm.at[pl.ds((off + base) // _CH, 1)], dstv)

            @pl.when(cid == 0)
            def _():
                pltpu.sync_copy(z_hbm.at[0, pl.ds(base, _CH)], rows)

            @pl.when(cid == 1)
            def _():
                pltpu.sync_copy(z_hbm.at[1, pl.ds(base, _CH)], rows)

            pltpu.sync_copy(rows, acc.at[dstv.at[0]], add=True)

        plsc.subcore_barrier()
        out_rows = pl.ds(sid * _NRPS, _NRPS)

        @pl.when(cid == 0)
        def _():
            pltpu.sync_copy(acc.at[out_rows], m0_hbm.at[out_rows])

        @pl.when(cid == 1)
        def _():
            pltpu.sync_copy(acc.at[out_rows], m1_hbm.at[out_rows])

    return _scatter_add


def _mk_gather_pair(off, ln):
    epw = ln // NW
    nch = epw // _CH

    @functools.partial(
        pl.kernel,
        mesh=_vmesh,
        out_type=[
            jax.ShapeDtypeStruct((ln, ND), jnp.float32),
            jax.ShapeDtypeStruct((ln, ND), jnp.float32),
        ],
        scratch_types=[
            pltpu.VMEM((1, _CH), jnp.int32),
            pltpu.VMEM((1, _CH), jnp.int32),
            pltpu.VMEM((_CH, ND), jnp.float32),
            pltpu.VMEM((_CH, ND), jnp.float32),
        ],
    )
    def _gather_pair(nfu_hbm, src2d_hbm, dst2d_hbm, s_hbm, t_hbm,
                     srcv, dstv, rs, rt):
        wid = lax.axis_index("s") * NC + lax.axis_index("c")

        @pl.loop(0, nch)
        def _(k):
            crow = pl.ds((off + wid * epw) // _CH + k, 1)
            pltpu.sync_copy(src2d_hbm.at[crow], srcv)
            pltpu.sync_copy(dst2d_hbm.at[crow], dstv)
            pltpu.sync_copy(nfu_hbm.at[srcv.at[0]], rs)
            pltpu.sync_copy(nfu_hbm.at[dstv.at[0]], rt)
            sl = pl.ds(wid * epw + k * _CH, _CH)
            pltpu.sync_copy(rs, s_hbm.at[sl])
            pltpu.sync_copy(rt, t_hbm.at[sl])

    return _gather_pair


_sc_a = tuple(_mk_gather_compose(_POFF[p], _PLEN[p]) for p in range(3))
_sc_c = tuple(_mk_scatter_add(_POFF[p], _PLEN[p]) for p in range(3))
_sc_e = tuple(_mk_gather_pair(_POFF[p], _PLEN[p]) for p in range(3))


def _bdot(x, w):
    return jnp.dot(x.astype(jnp.bfloat16), w[...],
                   preferred_element_type=jnp.float32)


def _tc_sub_body(x2, er, ea, wa, wb, wc, b, w0p, o):
    acc = _bdot(x2[...], wa) + _bdot(er[...], wb) + _bdot(ea[...], wc)
    sub = jax.nn.sigmoid(acc + b[...])
    z = _bdot(sub, w0p)
    o[0] = z[:, :HP]
    o[1] = z[:, HP:]


def _tc_sub(p, x2, er, ea, wa, wb, wc, b, w0p):
    ln = _PLEN[p]
    ob = _POFF[p] // _TB
    full = lambda r, c: pl.BlockSpec((r, c), lambda i: (0, 0))
    return pl.pallas_call(
        _tc_sub_body,
        grid=(ln // _TB,),
        in_specs=[
            pl.BlockSpec((_TB, ND), lambda i: (i, 0)),
            pl.BlockSpec((_TB, ERD), lambda i: (ob + i, 0)),
            pl.BlockSpec((_TB, EAD), lambda i: (ob + i, 0)),
            full(ND, ACD), full(ERD, ACD), full(EAD, ACD), full(1, ACD),
            full(ACD, ZP),
        ],
        out_specs=pl.BlockSpec((2, _TB, HP), lambda i: (0, i, 0)),
        out_shape=jax.ShapeDtypeStruct((2, ln, HP), jnp.float32),
    )(x2, er, ea, wa, wb, wc, b, w0p)


_BN = 1000


def _tc_node_body(m0a, m0b, m0c, m1a, m1b, m1c, nf, s0, s1,
                  b0, w1, b1, w2, b2, w3, b3, ws, bs, wn, bn, o):
    p0 = m0a[...] + m0b[...] + m0c[...]
    p1 = m1a[...] + m1b[...] + m1c[...]
    hpre = jnp.dot(p0, s0[...], preferred_element_type=jnp.float32)
    hpre += jnp.dot(p1, s1[...], preferred_element_type=jnp.float32)
    h = jax.nn.relu(hpre + b0[...])
    h = jax.nn.relu(jnp.dot(h, w1[...], preferred_element_type=jnp.float32)
                    + b1[...])
    h = jax.nn.relu(jnp.dot(h, w2[...], preferred_element_type=jnp.float32)
                    + b2[...])
    h = jnp.dot(h, w3[...], preferred_element_type=jnp.float32) + b3[...]
    nfv = nf[...]
    z = jnp.dot(nfv, ws[...], preferred_element_type=jnp.float32) + bs[...]
    z += jnp.dot(h, wn[...], preferred_element_type=jnp.float32) + bn[...]
    o[...] = jax.nn.sigmoid(z) + nfv


def _tc_node(ms, nf, s0, s1, b0, w1, b1, w2, b2, w3, b3, ws, bs, wn, bn):
    full = lambda r, c: pl.BlockSpec((r, c), lambda i: (0, 0))
    mspec = pl.BlockSpec((_BN, HP), lambda i: (i, 0))
    return pl.pallas_call(
        _tc_node_body,
        grid=(N // _BN,),
        in_specs=[
            mspec, mspec, mspec, mspec, mspec, mspec,
            pl.BlockSpec((_BN, ND), lambda i: (i, 0)),
            full(HP, HD0), full(HP, HD0),
            full(1, HD0),
            full(HD0, 64), full(1, 64),
            full(64, 128), full(1, 128),
            full(128, ND), full(1, ND),
            full(ND, ND), full(1, ND),
            full(ND, ND), full(1, ND),
        ],
        out_specs=pl.BlockSpec((_BN, ND), lambda i: (i, 0)),
        out_shape=jax.ShapeDtypeStruct((N, ND), jnp.float32),
    )(*ms, nf, s0, s1, b0, w1, b1, w2, b2, w3, b3, ws, bs, wn, bn)


def _tc_edge_body(s, t, er, ea, ewa, ewb, ewc, eb, wa, wb, wc, b,
                  v0, c0, v1, c1, v2, c2, v3, c3, o):
    a1 = _bdot(s[...], ewa) + _bdot(er[...], ewb) + _bdot(ea[...], ewc)
    a2 = _bdot(t[...], wa) + _bdot(er[...], wb) + _bdot(ea[...], wc)
    g = jax.nn.sigmoid(a1 + eb[...]) + jax.nn.sigmoid(a2 + b[...])
    g = jax.nn.relu(_bdot(g, v0) + c0[...])
    g = jax.nn.relu(_bdot(g, v1) + c1[...])
    g = jax.nn.relu(_bdot(g, v2) + c2[...])
    o[...] = _bdot(g, v3) + c3[...]


def _tc_edge(p, s, t, er, ea, ewa, ewb, ewc, eb, wa, wb, wc, b,
             v0, c0, v1, c1, v2, c2, v3, c3):
    ln = _PLEN[p]
    ob = _POFF[p] // _TB
    full = lambda r, c: pl.BlockSpec((r, c), lambda i: (0, 0))
    return pl.pallas_call(
        _tc_edge_body,
        grid=(ln // _TB,),
        in_specs=[
            pl.BlockSpec((_TB, ND), lambda i: (i, 0)),
            pl.BlockSpec((_TB, ND), lambda i: (i, 0)),
            pl.BlockSpec((_TB, ERD), lambda i: (ob + i, 0)),
            pl.BlockSpec((_TB, EAD), lambda i: (ob + i, 0)),
            full(ND, ACD), full(ERD, ACD), full(EAD, ACD), full(1, ACD),
            full(ND, ACD), full(ERD, ACD), full(EAD, ACD), full(1, ACD),
            full(ACD, 148), full(1, 148),
            full(148, 8), full(1, 8),
            full(8, 16), full(1, 16),
            full(16, 32), full(1, 32),
        ],
        out_specs=pl.BlockSpec((_TB, 32), lambda i: (i, 0)),
        out_shape=jax.ShapeDtypeStruct((ln, 32), jnp.float32),
    )(s, t, er, ea, ewa, ewb, ewc, eb, wa, wb, wc, b,
      v0, c0, v1, c1, v2, c2, v3, c3)


def kernel(node_features, edge_radial, edge_angular, edge_index,
           mlp1_w, mlp1_b, mlp2_w0, mlp2_b0, mlp2_w1, mlp2_b1,
           mlp2_w2, mlp2_b2, mlp2_w3, mlp2_b3, self_w, self_b,
           neigh_w, neigh_b, emlp1_w, emlp1_b, emlp2_w0, emlp2_b0,
           emlp2_w1, emlp2_b1, emlp2_w2, emlp2_b2, emlp2_w3, emlp2_b3):
    src = edge_index[0]
    dst = edge_index[1]
    src2d = src.reshape(E // _CH, _CH)
    dst2d = dst.reshape(E // _CH, _CH)

    bf = jnp.bfloat16
    w1t = mlp1_w.T.astype(bf)
    wa, wb, wc = w1t[:ND], w1t[ND:ND + ERD], w1t[ND + ERD:]
    b1r = mlp1_b[None, :]
    e1t = emlp1_w.T.astype(bf)
    ewa, ewb, ewc = e1t[:ND], e1t[ND:ND + ERD], e1t[ND + ERD:]
    eb1r = emlp1_b[None, :]
    w0p = jnp.pad(mlp2_w0.T, ((0, 0), (0, ZP - HD0))).astype(bf)
    zeros = jnp.zeros((_NRPS, HP), jnp.float32)
    sel0 = jnp.eye(HP, HD0, dtype=jnp.float32)
    sel1 = jnp.eye(HP, HD0, k=HP, dtype=jnp.float32)

    ms = [None] * 6
    for p in range(3):
        x2 = _sc_a[p](node_features, src2d, src)
        z = _tc_sub(p, x2, edge_radial, edge_angular, wa, wb, wc, b1r, w0p)
        ms[p], ms[3 + p] = _sc_c[p](z, dst2d, zeros)

    nfu = _tc_node(ms, node_features, sel0, sel1,
                   mlp2_b0[None, :],
                   mlp2_w1.T, mlp2_b1[None, :],
                   mlp2_w2.T, mlp2_b2[None, :],
                   mlp2_w3.T, mlp2_b3[None, :],
                   self_w.T, self_b[None, :],
                   neigh_w.T, neigh_b[None, :])

    e_p = [None] * 3
    for p in range(3):
        s_rows, t_rows = _sc_e[p](nfu, src2d, dst2d)
        e_p[p] = _tc_edge(p, s_rows, t_rows, edge_radial, edge_angular,
                          ewa, ewb, ewc, eb1r, wa, wb, wc, b1r,
                          emlp2_w0.T.astype(bf), emlp2_b0[None, :],
                          emlp2_w1.T.astype(bf), emlp2_b1[None, :],
                          emlp2_w2.T.astype(bf), emlp2_b2[None, :],
                          emlp2_w3.T.astype(bf), emlp2_b3[None, :])

    return (nfu, jnp.concatenate(e_p, axis=0))
